# Initial kernel scaffold; baseline (speedup 1.0000x reference)
import jax
import jax.numpy as jnp
from jax.experimental import pallas as pl


def kernel(atom_in_fea, nbr_fea, nbr_fea_idx, W_fc, b_fc, bn1_gamma, bn1_beta, bn1_mean, bn1_var, bn2_gamma, bn2_beta, bn2_mean, bn2_var):
    raise NotImplementedError("write your pallas kernel here")


# speedup vs baseline: 3.0874x; 3.0874x over previous
import functools

import jax
import jax.numpy as jnp
from jax import lax
from jax.experimental import pallas as pl
from jax.experimental.pallas import tpu as pltpu
from jax.experimental.pallas import tpu_sc as plsc

_EPS = 1e-3
_NC = 2
_NS = 16
_NW = _NC * _NS


def _sc_gather(table, idx_flat, chunk=400):
    nm = idx_flat.shape[0]
    d = table.shape[1]
    per_w = nm // _NW
    n_ch = per_w // chunk
    assert per_w % chunk == 0 and chunk % 8 == 0 and nm % _NW == 0

    mesh = plsc.VectorSubcoreMesh(core_axis_name="c", subcore_axis_name="s")

    @functools.partial(
        pl.kernel,
        mesh=mesh,
        out_type=jax.ShapeDtypeStruct((nm, d), jnp.float32),
        scratch_types=[
            pltpu.VMEM((chunk,), jnp.int32),
            pltpu.VMEM((chunk, d), jnp.float32),
            pltpu.SemaphoreType.DMA,
        ],
    )
    def gather_kernel(table_hbm, idx_hbm, out_hbm, idx_v, rows_v, sem):
        wid = lax.axis_index("s") * _NC + lax.axis_index("c")
        base = wid * per_w

        def body(c, carry):
            off = base + c * chunk
            pltpu.sync_copy(idx_hbm.at[pl.ds(off, chunk)], idx_v)
            pltpu.async_copy(table_hbm.at[idx_v], rows_v, sem).wait()
            pltpu.sync_copy(rows_v, out_hbm.at[pl.ds(off, chunk)])
            return carry

        lax.fori_loop(0, n_ch, body, 0, unroll=False)

    return gather_kernel(table, idx_flat)


def _softplus(x):
    return jnp.maximum(x, 0.0) + jnp.log1p(jnp.exp(-jnp.abs(x)))


def _tc_body(m, a_len, atom_ref, g_ref, nb_ref, ws_ref, wn_ref, we_ref,
             b_ref, s2_ref, b2_ref, out_ref):
    a = atom_ref[...]
    s = jnp.dot(a, ws_ref[...], preferred_element_type=jnp.float32)
    s = s + b_ref[...]
    x = jnp.dot(g_ref[...], wn_ref[...], preferred_element_type=jnp.float32)
    x = x + jnp.dot(nb_ref[...], we_ref[...], preferred_element_type=jnp.float32)
    bsz = a.shape[0]
    x = x.reshape(bsz, m, 2 * a_len) + s[:, None, :]
    filt = 1.0 / (1.0 + jnp.exp(-x[:, :, :a_len]))
    core = _softplus(x[:, :, a_len:])
    red = jnp.sum(filt * core, axis=1)
    red = red * s2_ref[...] + b2_ref[...]
    out_ref[...] = _softplus(a + red)


def _tc_main(atom, g, nb_flat, ws, wn, we, bvec, s2, b2, block=400):
    n, a_len = atom.shape
    nm = g.shape[0]
    m = nm // n
    e_len = nb_flat.shape[1]
    assert n % block == 0
    grid = (n // block,)
    body = functools.partial(_tc_body, m, a_len)
    return pl.pallas_call(
        body,
        grid=grid,
        in_specs=[
            pl.BlockSpec((block, a_len), lambda i: (i, 0)),
            pl.BlockSpec((block * m, a_len), lambda i: (i, 0)),
            pl.BlockSpec((block * m, e_len), lambda i: (i, 0)),
            pl.BlockSpec((a_len, 2 * a_len), lambda i: (0, 0)),
            pl.BlockSpec((a_len, 2 * a_len), lambda i: (0, 0)),
            pl.BlockSpec((e_len, 2 * a_len), lambda i: (0, 0)),
            pl.BlockSpec((1, 2 * a_len), lambda i: (0, 0)),
            pl.BlockSpec((1, a_len), lambda i: (0, 0)),
            pl.BlockSpec((1, a_len), lambda i: (0, 0)),
        ],
        out_specs=pl.BlockSpec((block, a_len), lambda i: (i, 0)),
        out_shape=jax.ShapeDtypeStruct((n, a_len), jnp.float32),
        compiler_params=pltpu.CompilerParams(
            dimension_semantics=("arbitrary",),
        ),
    )(atom, g, nb_flat, ws, wn, we, bvec, s2, b2)


def kernel(atom_in_fea, nbr_fea, nbr_fea_idx, W_fc, b_fc,
           bn1_gamma, bn1_beta, bn1_mean, bn1_var,
           bn2_gamma, bn2_beta, bn2_mean, bn2_var):
    n, m = nbr_fea_idx.shape
    a_len = atom_in_fea.shape[1]

    scale1 = bn1_gamma * lax.rsqrt(bn1_var + _EPS)
    wp = W_fc * scale1[None, :]
    bp = b_fc * scale1 + (bn1_beta - bn1_mean * scale1)
    ws = wp[:a_len]
    wn = wp[a_len:2 * a_len]
    we = wp[2 * a_len:]
    scale2 = bn2_gamma * lax.rsqrt(bn2_var + _EPS)
    bias2 = bn2_beta - bn2_mean * scale2

    idx_flat = nbr_fea_idx.reshape(-1).astype(jnp.int32)
    g = _sc_gather(atom_in_fea, idx_flat)
    nb_flat = nbr_fea.reshape(n * m, -1)
    return _tc_main(atom_in_fea, g, nb_flat, ws, wn, we,
                    bp.reshape(1, -1), scale2.reshape(1, -1),
                    bias2.reshape(1, -1))
